# baseline (speedup 1.0000x reference)
import jax
import jax.numpy as jnp
from jax.experimental import pallas as pl
from jax.experimental.pallas import tpu as pltpu

B, N, H = 16, 512, 256
PB = 4


def _body(exit_s, raise_s, cs_s, sl_s, braise_s, bbranch_s,
          ip_ref, ti_ref, fi_ref,
          c0_ref, h0_ref, c1_ref, h1_ref, emb_ref,
          Wi0_ref, Wh0_ref, b0_ref, Wi1_ref, Wh1_ref, b1_ref,
          Wr_ref, Wb_ref,
          oc0_ref, oh0_ref, oc1_ref, oh1_ref, oip_ref):
    f32 = jnp.float32
    bf16 = jnp.bfloat16

    def sg(v):
        return 0.5 * jnp.tanh(0.5 * v) + 0.5

    def lstm(c, h, xin, Wi_ref, Wh_ref, b_ref):
        z = (jnp.dot(xin.astype(bf16), Wi_ref[...].astype(bf16),
                     preferred_element_type=f32)
             + jnp.dot(h.astype(bf16), Wh_ref[...].astype(bf16),
                       preferred_element_type=f32)
             + b_ref[...])
        i = z[:, 0:H]
        f = z[:, H:2 * H]
        g = z[:, 2 * H:3 * H]
        o = z[:, 3 * H:4 * H]
        new_c = sg(f) * c + sg(i) * jnp.tanh(g)
        new_h = sg(o) * jnp.tanh(new_c)
        return new_c, new_h

    wd = jnp.concatenate([Wr_ref[:, 0:1] - Wr_ref[:, 1:2],
                          Wb_ref[:, 0:1] - Wb_ref[:, 1:2]],
                         axis=1).astype(bf16)
    rows = jax.lax.broadcasted_iota(jnp.int32, (N, 1), 0)
    std = (((1,), (0,)), ((), ()))
    ones8 = jnp.ones((N, 8), bf16)
    ones8r = jnp.ones((8, N), bf16)

    for k in range(PB):
        b = pl.program_id(0) * PB + k
        x = emb_ref[k]
        c0 = c0_ref[k]
        h0 = h0_ref[k]
        c1 = c1_ref[k]
        h1 = h1_ref[k]

        nc0, nh0 = lstm(c0, h0, x, Wi0_ref, Wh0_ref, b0_ref)
        nc1, nh1 = lstm(c1, h1, nh0, Wi1_ref, Wh1_ref, b1_ref)

        exit_i = exit_s[b]
        raise_i = raise_s[b]
        keep = (rows == exit_i) | (rows == raise_i)
        nc0 = jnp.where(keep, c0, nc0)
        nh0 = jnp.where(keep, h0, nh0)
        nc1 = jnp.where(keep, c1, nc1)
        nh1 = jnp.where(keep, h1, nh1)

        hcat = jnp.concatenate([nc0.astype(bf16), nh0.astype(bf16),
                                nc1.astype(bf16), nh1.astype(bf16)], axis=1)

        logits_t = jax.lax.dot_general(wd, hcat, (((0,), (1,)), ((), ())),
                                       preferred_element_type=f32)
        p_raise = sg(logits_t[0:1, :] + (braise_s[0] - braise_s[1]))
        p_tf = sg(logits_t[1:2, :] + (bbranch_s[0] - bbranch_s[1]))
        p_noraise = 1.0 - p_raise

        ipr = ip_ref[pl.ds(b, 1), :]
        wt = p_noraise * p_tf * ipr
        wf = p_noraise * (1.0 - p_tf) * ipr
        wr = p_raise * ipr

        ti = ti_ref[pl.ds(b, 1), :]
        fi = fi_ref[pl.ds(b, 1), :]
        zero = jnp.zeros((N, N), f32)
        mm = (jnp.where(ti == rows, jnp.broadcast_to(wt, (N, N)), zero)
              + jnp.where(fi == rows, jnp.broadcast_to(wf, (N, N)), zero)
              + jnp.where(rows == raise_i, jnp.broadcast_to(wr, (N, N)), zero)
              ).astype(bf16)

        agg = jax.lax.dot_general(mm, hcat, std, preferred_element_type=f32)
        den8 = jax.lax.dot_general(mm, ones8, std, preferred_element_type=f32)
        den = den8[:, 0:1]
        ipr8 = jax.lax.dot_general(ones8r, mm, (((1,), (1,)), ((), ())),
                                   preferred_element_type=f32)
        ip_new = ipr8[0:1, :]
        agg = agg * (1.0 / (den + 1e-7))

        pred = cs_s[b] < sl_s[b]
        oc0_ref[k] = jnp.where(pred, agg[:, 0:H], c0)
        oh0_ref[k] = jnp.where(pred, agg[:, H:2 * H], h0)
        oc1_ref[k] = jnp.where(pred, agg[:, 2 * H:3 * H], c1)
        oh1_ref[k] = jnp.where(pred, agg[:, 3 * H:4 * H], h1)
        oip_ref[pl.ds(b, 1), :] = jnp.where(pred, ip_new, ipr)


def kernel(c0, h0, c1, h1, instruction_pointer, current_step, node_embeddings,
           edge_sources, edge_dests, edge_types, true_indexes, false_indexes,
           exit_indexes, raise_indexes, step_limits,
           Wi0, Wh0, b0, Wi1, Wh1, b1, W_raise, b_raise, W_branch, b_branch):
    f32 = jnp.float32
    bnh = pl.BlockSpec((PB, N, H), lambda b, *_: (b, 0, 0))

    def const(shape):
        nd = len(shape)
        return pl.BlockSpec(shape, lambda b, *_: (0,) * nd)

    grid_spec = pltpu.PrefetchScalarGridSpec(
        num_scalar_prefetch=6,
        grid=(B // PB,),
        in_specs=[
            const((B, N)), const((B, N)), const((B, N)),
            bnh, bnh, bnh, bnh, bnh,
            const((H, 4 * H)), const((H, 4 * H)), const((4 * H,)),
            const((H, 4 * H)), const((H, 4 * H)), const((4 * H,)),
            const((4 * H, 2)), const((4 * H, 2)),
        ],
        out_specs=[bnh, bnh, bnh, bnh, const((B, N))],
    )

    out = pl.pallas_call(
        _body,
        grid_spec=grid_spec,
        out_shape=[
            jax.ShapeDtypeStruct((B, N, H), f32),
            jax.ShapeDtypeStruct((B, N, H), f32),
            jax.ShapeDtypeStruct((B, N, H), f32),
            jax.ShapeDtypeStruct((B, N, H), f32),
            jax.ShapeDtypeStruct((B, N), f32),
        ],
        compiler_params=pltpu.CompilerParams(
            dimension_semantics=("arbitrary",),
        ),
    )(exit_indexes, raise_indexes, current_step, step_limits,
      b_raise, b_branch,
      instruction_pointer, true_indexes, false_indexes,
      c0, h0, c1, h1, node_embeddings,
      Wi0, Wh0, b0, Wi1, Wh1, b1, W_raise, W_branch)

    oc0, oh0, oc1, oh1, oip = out
    return (oc0, oh0, oc1, oh1, oip, current_step + 1)


# speedup vs baseline: 1.0447x; 1.0447x over previous
import jax
import jax.numpy as jnp
from jax.experimental import pallas as pl
from jax.experimental.pallas import tpu as pltpu

B, N, H = 16, 512, 256
PB = 2


def _body(exit_s, raise_s, cs_s, sl_s, braise_s, bbranch_s,
          ip_ref, ti_ref, fi_ref,
          c0_ref, h0_ref, c1_ref, h1_ref, emb_ref,
          Wi0_ref, Wh0_ref, b0_ref, Wi1_ref, Wh1_ref, b1_ref,
          Wr_ref, Wb_ref,
          oc0_ref, oh0_ref, oc1_ref, oh1_ref, oip_ref):
    f32 = jnp.float32
    bf16 = jnp.bfloat16

    def sg(v):
        return 0.5 * jnp.tanh(0.5 * v) + 0.5

    def lstm(c, h, xin, Wi_ref, Wh_ref, b_ref):
        z = (jnp.dot(xin.astype(bf16), Wi_ref[...].astype(bf16),
                     preferred_element_type=f32)
             + jnp.dot(h.astype(bf16), Wh_ref[...].astype(bf16),
                       preferred_element_type=f32)
             + b_ref[...])
        i = z[:, 0:H]
        f = z[:, H:2 * H]
        g = z[:, 2 * H:3 * H]
        o = z[:, 3 * H:4 * H]
        new_c = sg(f) * c + sg(i) * jnp.tanh(g)
        new_h = sg(o) * jnp.tanh(new_c)
        return new_c, new_h

    wd = jnp.concatenate([Wr_ref[:, 0:1] - Wr_ref[:, 1:2],
                          Wb_ref[:, 0:1] - Wb_ref[:, 1:2]],
                         axis=1).astype(bf16)
    rows = jax.lax.broadcasted_iota(jnp.int32, (N, 1), 0)
    std = (((1,), (0,)), ((), ()))
    ones8 = jnp.ones((N, 8), bf16)
    ones8r = jnp.ones((8, N), bf16)

    for k in range(PB):
        b = pl.program_id(0) * PB + k
        x = emb_ref[k]
        c0 = c0_ref[k]
        h0 = h0_ref[k]
        c1 = c1_ref[k]
        h1 = h1_ref[k]

        nc0, nh0 = lstm(c0, h0, x, Wi0_ref, Wh0_ref, b0_ref)
        nc1, nh1 = lstm(c1, h1, nh0, Wi1_ref, Wh1_ref, b1_ref)

        exit_i = exit_s[b]
        raise_i = raise_s[b]
        keep = (rows == exit_i) | (rows == raise_i)
        nc0 = jnp.where(keep, c0, nc0)
        nh0 = jnp.where(keep, h0, nh0)
        nc1 = jnp.where(keep, c1, nc1)
        nh1 = jnp.where(keep, h1, nh1)

        hcat = jnp.concatenate([nc0.astype(bf16), nh0.astype(bf16),
                                nc1.astype(bf16), nh1.astype(bf16)], axis=1)

        logits_t = jax.lax.dot_general(wd, hcat, (((0,), (1,)), ((), ())),
                                       preferred_element_type=f32)
        p_raise = sg(logits_t[0:1, :] + (braise_s[0] - braise_s[1]))
        p_tf = sg(logits_t[1:2, :] + (bbranch_s[0] - bbranch_s[1]))
        p_noraise = 1.0 - p_raise

        ipr = ip_ref[pl.ds(b, 1), :]
        wt = p_noraise * p_tf * ipr
        wf = p_noraise * (1.0 - p_tf) * ipr
        wr = p_raise * ipr

        ti = ti_ref[pl.ds(b, 1), :]
        fi = fi_ref[pl.ds(b, 1), :]
        zero = jnp.zeros((N, N), f32)
        mm = (jnp.where(ti == rows, jnp.broadcast_to(wt, (N, N)), zero)
              + jnp.where(fi == rows, jnp.broadcast_to(wf, (N, N)), zero)
              + jnp.where(rows == raise_i, jnp.broadcast_to(wr, (N, N)), zero)
              ).astype(bf16)

        agg = jax.lax.dot_general(mm, hcat, std, preferred_element_type=f32)
        den8 = jax.lax.dot_general(mm, ones8, std, preferred_element_type=f32)
        den = den8[:, 0:1]
        ipr8 = jax.lax.dot_general(ones8r, mm, (((1,), (1,)), ((), ())),
                                   preferred_element_type=f32)
        ip_new = ipr8[0:1, :]
        agg = agg * (1.0 / (den + 1e-7))

        pred = cs_s[b] < sl_s[b]
        oc0_ref[k] = jnp.where(pred, agg[:, 0:H], c0)
        oh0_ref[k] = jnp.where(pred, agg[:, H:2 * H], h0)
        oc1_ref[k] = jnp.where(pred, agg[:, 2 * H:3 * H], c1)
        oh1_ref[k] = jnp.where(pred, agg[:, 3 * H:4 * H], h1)
        oip_ref[pl.ds(b, 1), :] = jnp.where(pred, ip_new, ipr)


def kernel(c0, h0, c1, h1, instruction_pointer, current_step, node_embeddings,
           edge_sources, edge_dests, edge_types, true_indexes, false_indexes,
           exit_indexes, raise_indexes, step_limits,
           Wi0, Wh0, b0, Wi1, Wh1, b1, W_raise, b_raise, W_branch, b_branch):
    f32 = jnp.float32
    bnh = pl.BlockSpec((PB, N, H), lambda b, *_: (b, 0, 0))

    def const(shape):
        nd = len(shape)
        return pl.BlockSpec(shape, lambda b, *_: (0,) * nd)

    grid_spec = pltpu.PrefetchScalarGridSpec(
        num_scalar_prefetch=6,
        grid=(B // PB,),
        in_specs=[
            const((B, N)), const((B, N)), const((B, N)),
            bnh, bnh, bnh, bnh, bnh,
            const((H, 4 * H)), const((H, 4 * H)), const((4 * H,)),
            const((H, 4 * H)), const((H, 4 * H)), const((4 * H,)),
            const((4 * H, 2)), const((4 * H, 2)),
        ],
        out_specs=[bnh, bnh, bnh, bnh, const((B, N))],
    )

    out = pl.pallas_call(
        _body,
        grid_spec=grid_spec,
        out_shape=[
            jax.ShapeDtypeStruct((B, N, H), f32),
            jax.ShapeDtypeStruct((B, N, H), f32),
            jax.ShapeDtypeStruct((B, N, H), f32),
            jax.ShapeDtypeStruct((B, N, H), f32),
            jax.ShapeDtypeStruct((B, N), f32),
        ],
        compiler_params=pltpu.CompilerParams(
            dimension_semantics=("arbitrary",),
        ),
    )(exit_indexes, raise_indexes, current_step, step_limits,
      b_raise, b_branch,
      instruction_pointer, true_indexes, false_indexes,
      c0, h0, c1, h1, node_embeddings,
      Wi0, Wh0, b0, Wi1, Wh1, b1, W_raise, W_branch)

    oc0, oh0, oc1, oh1, oip = out
    return (oc0, oh0, oc1, oh1, oip, current_step + 1)
